# baseline (device time: 105048 ns/iter reference)
import jax
import jax.numpy as jnp
from jax import lax
from jax.experimental import pallas as pl
from jax.experimental.pallas import tpu as pltpu

N_DEV = 4
B, S, D = 2, 512, 2048
H, DH, DR = 16, 128, 32
DC = 512
DCL = DC // N_DEV
BS = B * S
SCALE = (DH + DR) ** -0.5
NC, CW = 16, 128
NS = 4
NA = 24

_DevId = getattr(pl, "DeviceIdType", None) or pltpu.DeviceIdType
_sem_signal = getattr(pl, "semaphore_signal", None) or pltpu.semaphore_signal
_sem_wait = getattr(pl, "semaphore_wait", None) or pltpu.semaphore_wait
_CompilerParams = getattr(pltpu, "CompilerParams", None) or pltpu.TPUCompilerParams

CT, KU, VU = 0, 1, 2


def _dot(a, b, dims):
    return lax.dot_general(a, b, (dims, ((), ())),
                           preferred_element_type=jnp.float32)


def kernel(x, Wdkv, Wuk, Wuv, Wq, Wqr, Wkr, Wo):
    f = jnp.bfloat16

    def body(x_ref, wdkv_ref, wuk_ref, wuv_ref, wq_ref, wqr_ref, wkr_ref,
             wo_ref, out_ref,
             ct_ref, ku_ref, vu_ref, q_ref, qr_ref, kr_ref,
             k_ref, v_ref, o_ref, e_ref, d_ref, wstage_ref,
             send_sems, recv_sems, dma_sems):
        my = lax.axis_index("i")
        others = [(my + d) % N_DEV for d in (1, 2, 3)]

        barrier = pltpu.get_barrier_semaphore()
        for nbr in others:
            _sem_signal(barrier, inc=1, device_id=(nbr,),
                        device_id_type=_DevId.MESH)
        _sem_wait(barrier, 3)

        xv = x_ref[...].reshape(BS, D).astype(f)
        ct = _dot(wdkv_ref[...].astype(f), xv, ((0,), (1,))).astype(f)
        myrows = pl.ds(my * DCL, DCL)
        ct_ref[myrows] = ct
        ku_ref[myrows] = wuk_ref[...].astype(f)
        vu_ref[myrows] = wuv_ref[...].astype(f)

        def push(t, buf, dest):
            r = pltpu.make_async_remote_copy(
                src_ref=buf.at[myrows],
                dst_ref=buf.at[myrows],
                send_sem=send_sems.at[t, dest],
                recv_sem=recv_sems.at[t, my],
                device_id=(dest,),
                device_id_type=_DevId.MESH,
            )
            r.start()
            return r

        def drain(t, buf, origin):
            orows = pl.ds(origin * DCL, DCL)
            pltpu.make_async_remote_copy(
                src_ref=buf.at[orows],
                dst_ref=buf.at[orows],
                send_sem=send_sems.at[t, origin],
                recv_sem=recv_sems.at[t, origin],
                device_id=(my,),
                device_id_type=_DevId.MESH,
            ).wait_recv()

        sends = []
        for dest in others:
            sends.append(push(CT, ct_ref, dest))
        for dest in others:
            sends.append(push(KU, ku_ref, dest))

        def stream_chunk(w_hbm, ci, slot):
            cp = pltpu.make_async_copy(
                w_hbm.at[:, pl.ds(ci * CW, CW)],
                wstage_ref.at[slot],
                dma_sems.at[slot],
            )
            cp.start()
            return cp

        cps = [stream_chunk(wq_ref, ci, ci) for ci in range(NS)]
        for ci in range(NC):
            slot = ci % NS
            cps[slot].wait()
            wb = wstage_ref[slot].astype(f)
            if ci + NS < NC:
                cps[slot] = stream_chunk(wq_ref, ci + NS, slot)
            q_ref[:, ci * CW:(ci + 1) * CW] = (
                _dot(xv, wb, ((1,), (0,))) * SCALE).astype(f)
        qr_ref[...] = (_dot(xv, wqr_ref[...].astype(f), ((1,), (0,)))
                       * SCALE).astype(f)
        kr_ref[...] = _dot(xv, wkr_ref[...].astype(f), ((1,), (0,))).astype(f)

        for dest in others:
            sends.append(push(VU, vu_ref, dest))

        cps = [stream_chunk(wo_ref, ci, ci) for ci in range(NS)]

        for o in others:
            drain(CT, ct_ref, o)
        for o in others:
            drain(KU, ku_ref, o)
        k_ref[...] = _dot(ct_ref[...], ku_ref[...], ((0,), (0,))).astype(f)

        def scores_exp(b, hh):
            rows = slice(b * S, (b + 1) * S)
            cols = slice(hh * DH, (hh + 1) * DH)
            rcols = slice(hh * DR, (hh + 1) * DR)
            s = _dot(q_ref[rows, cols], k_ref[rows, cols], ((1,), (1,)))
            s = s + _dot(qr_ref[rows, rcols], kr_ref[rows, :], ((1,), (1,)))
            return jnp.exp(s.astype(f))

        def _recip_rowsum(e):
            return 1.0 / jnp.sum(e, axis=-1, keepdims=True,
                                 dtype=jnp.float32)

        def pv(e, recip, b, hh):
            rows = slice(b * S, (b + 1) * S)
            cols = slice(hh * DH, (hh + 1) * DH)
            o = _dot(e, v_ref[rows, cols], ((1,), (0,)))
            o_ref[rows, cols] = (o * recip).astype(f)

        pairs = [(b, hh) for b in range(B) for hh in range(H)]
        for i, (b, hh) in enumerate(pairs[:NA]):
            e = scores_exp(b, hh)
            e_ref[i * S:(i + 1) * S, :] = e
            d_ref[:, i:i + 1] = _recip_rowsum(e)

        for o in others:
            drain(VU, vu_ref, o)
        v_ref[...] = _dot(ct_ref[...], vu_ref[...], ((0,), (0,))).astype(f)

        for i, (b, hh) in enumerate(pairs[:NA]):
            pv(e_ref[i * S:(i + 1) * S, :], d_ref[:, i:i + 1], b, hh)
        for (b, hh) in pairs[NA:]:
            e = scores_exp(b, hh)
            pv(e, _recip_rowsum(e), b, hh)

        for ci in range(NC):
            slot = ci % NS
            cps[slot].wait()
            wb = wstage_ref[slot].astype(f)
            if ci + NS < NC:
                cps[slot] = stream_chunk(wo_ref, ci + NS, slot)
            cols = slice(ci * CW, (ci + 1) * CW)
            out_ref[0, :, cols] = _dot(o_ref[0:S, :], wb,
                                       ((1,), (0,))).astype(f)
            out_ref[1, :, cols] = _dot(o_ref[S:BS, :], wb,
                                       ((1,), (0,))).astype(f)

        for r in sends:
            r.wait_send()

    vmem = pl.BlockSpec(memory_space=pltpu.VMEM)
    anym = pl.BlockSpec(memory_space=pl.ANY)
    return pl.pallas_call(
        body,
        out_shape=jax.ShapeDtypeStruct((B, S, D), f),
        in_specs=[vmem, vmem, vmem, vmem, anym, vmem, vmem, anym],
        out_specs=vmem,
        scratch_shapes=[
            pltpu.VMEM((DC, BS), f),
            pltpu.VMEM((DC, D), f),
            pltpu.VMEM((DC, D), f),
            pltpu.VMEM((BS, D), f),
            pltpu.VMEM((BS, H * DR), f),
            pltpu.VMEM((BS, DR), f),
            pltpu.VMEM((BS, D), f),
            pltpu.VMEM((BS, D), f),
            pltpu.VMEM((BS, D), f),
            pltpu.VMEM((NA * S, S), f),
            pltpu.VMEM((S, NA), jnp.float32),
            pltpu.VMEM((NS, D, CW), jnp.float32),
            pltpu.SemaphoreType.DMA((3, N_DEV)),
            pltpu.SemaphoreType.DMA((3, N_DEV)),
            pltpu.SemaphoreType.DMA((NS,)),
        ],
        compiler_params=_CompilerParams(
            collective_id=0, vmem_limit_bytes=64 * 1024 * 1024),
    )(x, Wdkv, Wuk, Wuv, Wq, Wqr, Wkr, Wo)


# device time: 78877 ns/iter; 1.3318x vs baseline; 1.3318x over previous
import jax
import jax.numpy as jnp
from jax import lax
from jax.experimental import pallas as pl
from jax.experimental.pallas import tpu as pltpu

N_DEV = 4
B, S, D = 2, 512, 2048
H, DH, DR = 16, 128, 32
DC = 512
DCL = DC // N_DEV
BS = B * S
SCALE = (DH + DR) ** -0.5
NC, CW = 8, 256
NS = 3
NA = 16

_DevId = getattr(pl, "DeviceIdType", None) or pltpu.DeviceIdType
_sem_signal = getattr(pl, "semaphore_signal", None) or pltpu.semaphore_signal
_sem_wait = getattr(pl, "semaphore_wait", None) or pltpu.semaphore_wait
_CompilerParams = getattr(pltpu, "CompilerParams", None) or pltpu.TPUCompilerParams

CT, KU, VU = 0, 1, 2


def _dot(a, b, dims):
    return lax.dot_general(a, b, (dims, ((), ())),
                           preferred_element_type=jnp.float32)


def kernel(x, Wdkv, Wuk, Wuv, Wq, Wqr, Wkr, Wo):
    f = jnp.bfloat16

    def body(x_ref, wdkv_ref, wuk_ref, wuv_ref, wq_ref, wqr_ref, wkr_ref,
             wo_ref, out_ref,
             ct_ref, ku_ref, vu_ref, q_ref, qr_ref, kr_ref,
             k_ref, v_ref, o_ref, e_ref, d_ref, wstage_ref,
             send_sems, recv_sems, dma_sems):
        my = lax.axis_index("i")
        others = [(my + d) % N_DEV for d in (1, 2, 3)]

        barrier = pltpu.get_barrier_semaphore()
        for nbr in others:
            _sem_signal(barrier, inc=1, device_id=(nbr,),
                        device_id_type=_DevId.MESH)
        _sem_wait(barrier, 3)

        xv = x_ref[...].reshape(BS, D).astype(f)
        ct = _dot(wdkv_ref[...].astype(f), xv, ((0,), (1,))).astype(f)
        myrows = pl.ds(my * DCL, DCL)
        ct_ref[myrows] = ct
        ku_ref[myrows] = wuk_ref[...].astype(f)
        vu_ref[myrows] = wuv_ref[...].astype(f)

        def push(t, buf, dest):
            r = pltpu.make_async_remote_copy(
                src_ref=buf.at[myrows],
                dst_ref=buf.at[myrows],
                send_sem=send_sems.at[t, dest],
                recv_sem=recv_sems.at[t, my],
                device_id=(dest,),
                device_id_type=_DevId.MESH,
            )
            r.start()
            return r

        def drain(t, buf, origin):
            orows = pl.ds(origin * DCL, DCL)
            pltpu.make_async_remote_copy(
                src_ref=buf.at[orows],
                dst_ref=buf.at[orows],
                send_sem=send_sems.at[t, origin],
                recv_sem=recv_sems.at[t, origin],
                device_id=(my,),
                device_id_type=_DevId.MESH,
            ).wait_recv()

        sends = []
        for dest in others:
            sends.append(push(CT, ct_ref, dest))
        for dest in others:
            sends.append(push(KU, ku_ref, dest))

        def stream_chunk(w_hbm, ci, slot):
            cp = pltpu.make_async_copy(
                w_hbm.at[:, pl.ds(ci * CW, CW)],
                wstage_ref.at[slot],
                dma_sems.at[slot],
            )
            cp.start()
            return cp

        cps = [stream_chunk(wq_ref, ci, ci) for ci in range(NS)]
        for ci in range(NC):
            slot = ci % NS
            cps[slot].wait()
            wb = wstage_ref[slot].astype(f)
            if ci + NS < NC:
                cps[slot] = stream_chunk(wq_ref, ci + NS, slot)
            q_ref[:, ci * CW:(ci + 1) * CW] = (
                _dot(xv, wb, ((1,), (0,))) * SCALE).astype(f)
        qr_ref[...] = (_dot(xv, wqr_ref[...].astype(f), ((1,), (0,)))
                       * SCALE).astype(f)
        kr_ref[...] = _dot(xv, wkr_ref[...].astype(f), ((1,), (0,))).astype(f)

        for dest in others:
            sends.append(push(VU, vu_ref, dest))

        cps = [stream_chunk(wo_ref, ci, ci) for ci in range(NS)]

        for o in others:
            drain(CT, ct_ref, o)
        for o in others:
            drain(KU, ku_ref, o)
        k_ref[...] = _dot(ct_ref[...], ku_ref[...], ((0,), (0,))).astype(f)

        def scores_exp(b, hh):
            rows = slice(b * S, (b + 1) * S)
            cols = slice(hh * DH, (hh + 1) * DH)
            rcols = slice(hh * DR, (hh + 1) * DR)
            s = _dot(q_ref[rows, cols], k_ref[rows, cols], ((1,), (1,)))
            s = s + _dot(qr_ref[rows, rcols], kr_ref[rows, :], ((1,), (1,)))
            return jnp.exp(s.astype(f))

        def _recip_rowsum(e):
            return 1.0 / jnp.sum(e, axis=-1, keepdims=True,
                                 dtype=jnp.float32)

        def pv(e, recip, b, hh):
            rows = slice(b * S, (b + 1) * S)
            cols = slice(hh * DH, (hh + 1) * DH)
            o = _dot(e, v_ref[rows, cols], ((1,), (0,)))
            o_ref[rows, cols] = (o * recip).astype(f)

        pairs = [(b, hh) for b in range(B) for hh in range(H)]
        for i, (b, hh) in enumerate(pairs[:NA]):
            e = scores_exp(b, hh)
            e_ref[i * S:(i + 1) * S, :] = e
            d_ref[:, i:i + 1] = _recip_rowsum(e)

        for o in others:
            drain(VU, vu_ref, o)
        v_ref[...] = _dot(ct_ref[...], vu_ref[...], ((0,), (0,))).astype(f)

        for i, (b, hh) in enumerate(pairs[:NA]):
            pv(e_ref[i * S:(i + 1) * S, :], d_ref[:, i:i + 1], b, hh)
        for (b, hh) in pairs[NA:]:
            e = scores_exp(b, hh)
            pv(e, _recip_rowsum(e), b, hh)

        for ci in range(NC):
            slot = ci % NS
            cps[slot].wait()
            wb = wstage_ref[slot].astype(f)
            if ci + NS < NC:
                cps[slot] = stream_chunk(wo_ref, ci + NS, slot)
            cols = slice(ci * CW, (ci + 1) * CW)
            out_ref[0, :, cols] = _dot(o_ref[0:S, :], wb,
                                       ((1,), (0,))).astype(f)
            out_ref[1, :, cols] = _dot(o_ref[S:BS, :], wb,
                                       ((1,), (0,))).astype(f)

        for r in sends:
            r.wait_send()

    vmem = pl.BlockSpec(memory_space=pltpu.VMEM)
    anym = pl.BlockSpec(memory_space=pl.ANY)
    return pl.pallas_call(
        body,
        out_shape=jax.ShapeDtypeStruct((B, S, D), f),
        in_specs=[vmem, vmem, vmem, vmem, anym, vmem, vmem, anym],
        out_specs=vmem,
        scratch_shapes=[
            pltpu.VMEM((DC, BS), f),
            pltpu.VMEM((DC, D), f),
            pltpu.VMEM((DC, D), f),
            pltpu.VMEM((BS, D), f),
            pltpu.VMEM((BS, H * DR), f),
            pltpu.VMEM((BS, DR), f),
            pltpu.VMEM((BS, D), f),
            pltpu.VMEM((BS, D), f),
            pltpu.VMEM((BS, D), f),
            pltpu.VMEM((NA * S, S), f),
            pltpu.VMEM((S, NA), jnp.float32),
            pltpu.VMEM((NS, D, CW), jnp.float32),
            pltpu.SemaphoreType.DMA((3, N_DEV)),
            pltpu.SemaphoreType.DMA((3, N_DEV)),
            pltpu.SemaphoreType.DMA((NS,)),
        ],
        compiler_params=_CompilerParams(
            collective_id=0, vmem_limit_bytes=64 * 1024 * 1024),
    )(x, Wdkv, Wuk, Wuv, Wq, Wqr, Wkr, Wo)


# device time: 77562 ns/iter; 1.3544x vs baseline; 1.0170x over previous
import jax
import jax.numpy as jnp
from jax import lax
from jax.experimental import pallas as pl
from jax.experimental.pallas import tpu as pltpu

N_DEV = 4
B, S, D = 2, 512, 2048
H, DH, DR = 16, 128, 32
DC = 512
DCL = DC // N_DEV
BS = B * S
SCALE = (DH + DR) ** -0.5
NC, CW = 8, 256
NS = 3
NA = 20

_DevId = getattr(pl, "DeviceIdType", None) or pltpu.DeviceIdType
_sem_signal = getattr(pl, "semaphore_signal", None) or pltpu.semaphore_signal
_sem_wait = getattr(pl, "semaphore_wait", None) or pltpu.semaphore_wait
_CompilerParams = getattr(pltpu, "CompilerParams", None) or pltpu.TPUCompilerParams

CT, KU, VU = 0, 1, 2


def _dot(a, b, dims):
    return lax.dot_general(a, b, (dims, ((), ())),
                           preferred_element_type=jnp.float32)


def kernel(x, Wdkv, Wuk, Wuv, Wq, Wqr, Wkr, Wo):
    f = jnp.bfloat16

    def body(x_ref, wdkv_ref, wuk_ref, wuv_ref, wq_ref, wqr_ref, wkr_ref,
             wo_ref, out_ref,
             ct_ref, ku_ref, vu_ref, q_ref, qr_ref, kr_ref,
             k_ref, v_ref, o_ref, e_ref, d_ref, wstage_ref,
             send_sems, recv_sems, dma_sems):
        my = lax.axis_index("i")
        others = [(my + d) % N_DEV for d in (1, 2, 3)]

        barrier = pltpu.get_barrier_semaphore()
        for nbr in others:
            _sem_signal(barrier, inc=1, device_id=(nbr,),
                        device_id_type=_DevId.MESH)
        _sem_wait(barrier, 3)

        xv = x_ref[...].astype(f)
        ct = _dot(wdkv_ref[...].astype(f), xv, ((0,), (1,))).astype(f)
        myrows = pl.ds(my * DCL, DCL)
        ct_ref[myrows] = ct
        ku_ref[myrows] = wuk_ref[...].astype(f)
        vu_ref[myrows] = wuv_ref[...].astype(f)

        def push(t, buf, dest):
            r = pltpu.make_async_remote_copy(
                src_ref=buf.at[myrows],
                dst_ref=buf.at[myrows],
                send_sem=send_sems.at[t, dest],
                recv_sem=recv_sems.at[t, my],
                device_id=(dest,),
                device_id_type=_DevId.MESH,
            )
            r.start()
            return r

        def drain(t, buf, origin):
            orows = pl.ds(origin * DCL, DCL)
            pltpu.make_async_remote_copy(
                src_ref=buf.at[orows],
                dst_ref=buf.at[orows],
                send_sem=send_sems.at[t, origin],
                recv_sem=recv_sems.at[t, origin],
                device_id=(my,),
                device_id_type=_DevId.MESH,
            ).wait_recv()

        sends = []
        for dest in others:
            sends.append(push(CT, ct_ref, dest))
        for dest in others:
            sends.append(push(KU, ku_ref, dest))

        def stream_chunk(w_hbm, ci, slot):
            cp = pltpu.make_async_copy(
                w_hbm.at[:, pl.ds(ci * CW, CW)],
                wstage_ref.at[slot],
                dma_sems.at[slot],
            )
            cp.start()
            return cp

        cps = [stream_chunk(wq_ref, ci, ci) for ci in range(NS)]
        for ci in range(NC):
            slot = ci % NS
            cps[slot].wait()
            wb = wstage_ref[slot].astype(f)
            if ci + NS < NC:
                cps[slot] = stream_chunk(wq_ref, ci + NS, slot)
            q_ref[:, ci * CW:(ci + 1) * CW] = (
                _dot(xv, wb, ((1,), (0,))) * SCALE).astype(f)
        qr_ref[...] = (_dot(xv, wqr_ref[...].astype(f), ((1,), (0,)))
                       * SCALE).astype(f)
        kr_ref[...] = _dot(xv, wkr_ref[...].astype(f), ((1,), (0,))).astype(f)

        for dest in others:
            sends.append(push(VU, vu_ref, dest))

        cps = [stream_chunk(wo_ref, ci, ci) for ci in range(NS)]

        for o in others:
            drain(CT, ct_ref, o)
        for o in others:
            drain(KU, ku_ref, o)
        k_ref[...] = _dot(ct_ref[...], ku_ref[...], ((0,), (0,))).astype(f)

        def scores_exp(b, hh):
            rows = slice(b * S, (b + 1) * S)
            cols = slice(hh * DH, (hh + 1) * DH)
            rcols = slice(hh * DR, (hh + 1) * DR)
            s = _dot(q_ref[rows, cols], k_ref[rows, cols], ((1,), (1,)))
            s = s + _dot(qr_ref[rows, rcols], kr_ref[rows, :], ((1,), (1,)))
            return jnp.exp(s.astype(f))

        def _recip_rowsum(e):
            return 1.0 / jnp.sum(e, axis=-1, keepdims=True,
                                 dtype=jnp.float32)

        def pv(e, recip, b, hh):
            rows = slice(b * S, (b + 1) * S)
            cols = slice(hh * DH, (hh + 1) * DH)
            o = _dot(e, v_ref[rows, cols], ((1,), (0,)))
            o_ref[rows, cols] = (o * recip).astype(f)

        pairs = [(b, hh) for b in range(B) for hh in range(H)]
        for i, (b, hh) in enumerate(pairs[:NA]):
            e = scores_exp(b, hh)
            e_ref[i * S:(i + 1) * S, :] = e
            d_ref[:, i:i + 1] = _recip_rowsum(e)

        for o in others:
            drain(VU, vu_ref, o)
        v_ref[...] = _dot(ct_ref[...], vu_ref[...], ((0,), (0,))).astype(f)

        for i, (b, hh) in enumerate(pairs[:NA]):
            pv(e_ref[i * S:(i + 1) * S, :], d_ref[:, i:i + 1], b, hh)
        for (b, hh) in pairs[NA:]:
            e = scores_exp(b, hh)
            pv(e, _recip_rowsum(e), b, hh)

        for ci in range(NC):
            slot = ci % NS
            cps[slot].wait()
            wb = wstage_ref[slot].astype(f)
            if ci + NS < NC:
                cps[slot] = stream_chunk(wo_ref, ci + NS, slot)
            cols = slice(ci * CW, (ci + 1) * CW)
            out_ref[0, :, cols] = _dot(o_ref[0:S, :], wb,
                                       ((1,), (0,))).astype(f)
            out_ref[1, :, cols] = _dot(o_ref[S:BS, :], wb,
                                       ((1,), (0,))).astype(f)

        for r in sends:
            r.wait_send()

    vmem = pl.BlockSpec(memory_space=pltpu.VMEM)
    anym = pl.BlockSpec(memory_space=pl.ANY)
    return pl.pallas_call(
        body,
        out_shape=jax.ShapeDtypeStruct((B, S, D), f),
        in_specs=[vmem, vmem, vmem, vmem, anym, vmem, vmem, anym],
        out_specs=vmem,
        scratch_shapes=[
            pltpu.VMEM((DC, BS), f),
            pltpu.VMEM((DC, D), f),
            pltpu.VMEM((DC, D), f),
            pltpu.VMEM((BS, D), f),
            pltpu.VMEM((BS, H * DR), f),
            pltpu.VMEM((BS, DR), f),
            pltpu.VMEM((BS, D), f),
            pltpu.VMEM((BS, D), f),
            pltpu.VMEM((BS, D), f),
            pltpu.VMEM((NA * S, S), f),
            pltpu.VMEM((S, NA), jnp.float32),
            pltpu.VMEM((NS, D, CW), jnp.float32),
            pltpu.SemaphoreType.DMA((3, N_DEV)),
            pltpu.SemaphoreType.DMA((3, N_DEV)),
            pltpu.SemaphoreType.DMA((NS,)),
        ],
        compiler_params=_CompilerParams(
            collective_id=0, vmem_limit_bytes=64 * 1024 * 1024),
    )(x.reshape(BS, D), Wdkv, Wuk, Wuv, Wq, Wqr, Wkr, Wo)


# device time: 71554 ns/iter; 1.4681x vs baseline; 1.0840x over previous
import jax
import jax.numpy as jnp
from jax import lax
from jax.experimental import pallas as pl
from jax.experimental.pallas import tpu as pltpu

N_DEV = 4
B, S, D = 2, 512, 2048
H, DH, DR = 16, 128, 32
DC = 512
DCL = DC // N_DEV
BS = B * S
HL = H // N_DEV
GW = HL * DH
GR = HL * DR
SCALE = (DH + DR) ** -0.5
NC, CW = 8, 256
NS = 4

_DevId = getattr(pl, "DeviceIdType", None) or pltpu.DeviceIdType
_sem_signal = getattr(pl, "semaphore_signal", None) or pltpu.semaphore_signal
_sem_wait = getattr(pl, "semaphore_wait", None) or pltpu.semaphore_wait
_CompilerParams = getattr(pltpu, "CompilerParams", None) or pltpu.TPUCompilerParams

CT, KU, VU = 0, 1, 2


def _dot(a, b, dims):
    return lax.dot_general(a, b, (dims, ((), ())),
                           preferred_element_type=jnp.float32)


def kernel(x, Wdkv, Wuk, Wuv, Wq, Wqr, Wkr, Wo):
    f = jnp.bfloat16

    def body(x_ref, wdkv_ref, wuk_ref, wuv_ref, wq_ref, wqr_ref, wkr_ref,
             wo_ref, out_ref,
             ct_ref, kup_ref, vup_ref, kusnd_ref, vusnd_ref,
             q_ref, qr_ref, kr_ref,
             kg_ref, vg_ref, o_ref, wstage_ref,
             send_sems, recv_sems, osend_sems, orecv_sems, dma_sems):
        my = lax.axis_index("i")
        others = [(my + d) % N_DEV for d in (1, 2, 3)]

        barrier = pltpu.get_barrier_semaphore()
        for nbr in others:
            _sem_signal(barrier, inc=1, device_id=(nbr,),
                        device_id_type=_DevId.MESH)
        _sem_wait(barrier, 3)

        xv = x_ref[...].astype(f)
        ct = _dot(wdkv_ref[...].astype(f), xv, ((0,), (1,))).astype(f)
        myrows = pl.ds(my * DCL, DCL)
        mygcols = pl.ds(my * GW, GW)
        ct_ref[myrows] = ct
        kusnd_ref[...] = wuk_ref[...].astype(f)
        vusnd_ref[...] = wuv_ref[...].astype(f)
        kup_ref[myrows] = kusnd_ref[:, mygcols]
        vup_ref[myrows] = vusnd_ref[:, mygcols]

        sends = []

        def push(t, src, dst, dest):
            r = pltpu.make_async_remote_copy(
                src_ref=src, dst_ref=dst,
                send_sem=send_sems.at[t, dest],
                recv_sem=recv_sems.at[t, my],
                device_id=(dest,), device_id_type=_DevId.MESH,
            )
            r.start()
            sends.append(r)

        def drain(t, buf, origin):
            orows = pl.ds(origin * DCL, DCL)
            pltpu.make_async_remote_copy(
                src_ref=buf.at[orows], dst_ref=buf.at[orows],
                send_sem=send_sems.at[t, origin],
                recv_sem=recv_sems.at[t, origin],
                device_id=(my,), device_id_type=_DevId.MESH,
            ).wait_recv()

        for dest in others:
            push(CT, ct_ref.at[myrows], ct_ref.at[myrows], dest)
        for dest in others:
            dcols = pl.ds(dest * GW, GW)
            push(KU, kusnd_ref.at[:, dcols], kup_ref.at[myrows], dest)
            push(VU, vusnd_ref.at[:, dcols], vup_ref.at[myrows], dest)

        def stream_chunk(w_hbm, off, slot):
            cp = pltpu.make_async_copy(
                w_hbm.at[:, pl.ds(off, CW)],
                wstage_ref.at[slot],
                dma_sems.at[slot],
            )
            cp.start()
            return cp

        cps = [stream_chunk(wq_ref, my * GW, 0),
               stream_chunk(wq_ref, my * GW + CW, 1)]
        for ci in range(2):
            cps[ci].wait()
            wb = wstage_ref[ci].astype(f)
            q_ref[:, ci * CW:(ci + 1) * CW] = (
                _dot(xv, wb, ((1,), (0,))) * SCALE).astype(f)
        myrcols = pl.ds(my * GR, GR)
        qr_ref[...] = (_dot(xv, wqr_ref[:, myrcols].astype(f), ((1,), (0,)))
                       * SCALE).astype(f)
        kr_ref[...] = _dot(xv, wkr_ref[...].astype(f), ((1,), (0,))).astype(f)

        cps = [stream_chunk(wo_ref, ci * CW, ci) for ci in range(NS)]

        for o in others:
            drain(CT, ct_ref, o)
        for o in others:
            drain(KU, kup_ref, o)
        kg_ref[...] = _dot(ct_ref[...], kup_ref[...], ((0,), (0,))).astype(f)
        for o in others:
            drain(VU, vup_ref, o)
        vg_ref[...] = _dot(ct_ref[...], vup_ref[...], ((0,), (0,))).astype(f)

        for b in range(B):
            rows = slice(b * S, (b + 1) * S)
            for lh in range(HL):
                cols = slice(lh * DH, (lh + 1) * DH)
                rcols = slice(lh * DR, (lh + 1) * DR)
                s = _dot(q_ref[rows, cols], kg_ref[rows, cols], ((1,), (1,)))
                s = s + _dot(qr_ref[rows, rcols], kr_ref[rows, :],
                             ((1,), (1,)))
                e = jnp.exp(s.astype(f))
                recip = 1.0 / jnp.sum(e, axis=-1, keepdims=True,
                                      dtype=jnp.float32)
                o = _dot(e, vg_ref[rows, cols], ((1,), (0,)))
                o_ref[rows, pl.ds(my * GW + lh * DH, DH)] = (
                    o * recip).astype(f)
            obs = o_ref.at[pl.ds(b * S, S), pl.ds(my * GW, GW)]
            for dest in others:
                r = pltpu.make_async_remote_copy(
                    src_ref=obs, dst_ref=obs,
                    send_sem=osend_sems.at[dest, b],
                    recv_sem=orecv_sems.at[my, b],
                    device_id=(dest,), device_id_type=_DevId.MESH,
                )
                r.start()
                sends.append(r)

        for o in others:
            for b in range(B):
                obs = o_ref.at[pl.ds(b * S, S), pl.ds(o * GW, GW)]
                pltpu.make_async_remote_copy(
                    src_ref=obs, dst_ref=obs,
                    send_sem=osend_sems.at[o, b],
                    recv_sem=orecv_sems.at[o, b],
                    device_id=(my,), device_id_type=_DevId.MESH,
                ).wait_recv()

        for ci in range(NC):
            slot = ci % NS
            cps[slot].wait()
            wb = wstage_ref[slot].astype(f)
            if ci + NS < NC:
                cps[slot] = stream_chunk(wo_ref, (ci + NS) * CW, slot)
            cols = slice(ci * CW, (ci + 1) * CW)
            out_ref[0, :, cols] = _dot(o_ref[0:S, :], wb,
                                       ((1,), (0,))).astype(f)
            out_ref[1, :, cols] = _dot(o_ref[S:BS, :], wb,
                                       ((1,), (0,))).astype(f)

        for r in sends:
            r.wait_send()

    vmem = pl.BlockSpec(memory_space=pltpu.VMEM)
    anym = pl.BlockSpec(memory_space=pl.ANY)
    return pl.pallas_call(
        body,
        out_shape=jax.ShapeDtypeStruct((B, S, D), f),
        in_specs=[vmem, vmem, vmem, vmem, anym, vmem, vmem, anym],
        out_specs=vmem,
        scratch_shapes=[
            pltpu.VMEM((DC, BS), f),
            pltpu.VMEM((DC, GW), f),
            pltpu.VMEM((DC, GW), f),
            pltpu.VMEM((DCL, D), f),
            pltpu.VMEM((DCL, D), f),
            pltpu.VMEM((BS, GW), f),
            pltpu.VMEM((BS, GR), f),
            pltpu.VMEM((BS, DR), f),
            pltpu.VMEM((BS, GW), f),
            pltpu.VMEM((BS, GW), f),
            pltpu.VMEM((BS, D), f),
            pltpu.VMEM((NS, D, CW), jnp.float32),
            pltpu.SemaphoreType.DMA((3, N_DEV)),
            pltpu.SemaphoreType.DMA((3, N_DEV)),
            pltpu.SemaphoreType.DMA((N_DEV, B)),
            pltpu.SemaphoreType.DMA((N_DEV, B)),
            pltpu.SemaphoreType.DMA((NS,)),
        ],
        compiler_params=_CompilerParams(
            collective_id=0, vmem_limit_bytes=64 * 1024 * 1024),
    )(x.reshape(BS, D), Wdkv, Wuk, Wuv, Wq, Wqr, Wkr, Wo)


# device time: 70905 ns/iter; 1.4815x vs baseline; 1.0092x over previous
import jax
import jax.numpy as jnp
from jax import lax
from jax.experimental import pallas as pl
from jax.experimental.pallas import tpu as pltpu

N_DEV = 4
B, S, D = 2, 512, 2048
H, DH, DR = 16, 128, 32
DC = 512
DCL = DC // N_DEV
BS = B * S
HL = H // N_DEV
GW = HL * DH
GR = HL * DR
SCALE = (DH + DR) ** -0.5
NC, CW = 8, 256
NS = 4

_DevId = getattr(pl, "DeviceIdType", None) or pltpu.DeviceIdType
_sem_signal = getattr(pl, "semaphore_signal", None) or pltpu.semaphore_signal
_sem_wait = getattr(pl, "semaphore_wait", None) or pltpu.semaphore_wait
_CompilerParams = getattr(pltpu, "CompilerParams", None) or pltpu.TPUCompilerParams

CT, KU, VU = 0, 1, 2


def _dot(a, b, dims):
    return lax.dot_general(a, b, (dims, ((), ())),
                           preferred_element_type=jnp.float32)


def kernel(x, Wdkv, Wuk, Wuv, Wq, Wqr, Wkr, Wo):
    f = jnp.bfloat16

    def body(x_ref, wdkv_ref, wuk_ref, wuv_ref, wq_ref, wqr_ref, wkr_ref,
             wo_ref, out_ref,
             ct_ref, kup_ref, vup_ref, kusnd_ref, vusnd_ref,
             q_ref, qr_ref, kr_ref,
             kg_ref, vg_ref, ot_ref, wob_ref, wstage_ref,
             send_sems, recv_sems, osend_sems, orecv_sems, dma_sems):
        my = lax.axis_index("i")
        others = [(my + d) % N_DEV for d in (1, 2, 3)]

        barrier = pltpu.get_barrier_semaphore()
        for nbr in others:
            _sem_signal(barrier, inc=1, device_id=(nbr,),
                        device_id_type=_DevId.MESH)
        _sem_wait(barrier, 3)

        xv = x_ref[...].astype(f)
        ct = _dot(wdkv_ref[...].astype(f), xv, ((0,), (1,))).astype(f)
        myrows = pl.ds(my * DCL, DCL)
        mygcols = pl.ds(my * GW, GW)
        ct_ref[myrows] = ct
        kusnd_ref[...] = wuk_ref[...].astype(f)
        vusnd_ref[...] = wuv_ref[...].astype(f)
        kup_ref[myrows] = kusnd_ref[:, mygcols]
        vup_ref[myrows] = vusnd_ref[:, mygcols]

        sends = []

        def push(t, src, dst, dest):
            r = pltpu.make_async_remote_copy(
                src_ref=src, dst_ref=dst,
                send_sem=send_sems.at[t, dest],
                recv_sem=recv_sems.at[t, my],
                device_id=(dest,), device_id_type=_DevId.MESH,
            )
            r.start()
            sends.append(r)

        def drain(t, buf, origin):
            orows = pl.ds(origin * DCL, DCL)
            pltpu.make_async_remote_copy(
                src_ref=buf.at[orows], dst_ref=buf.at[orows],
                send_sem=send_sems.at[t, origin],
                recv_sem=recv_sems.at[t, origin],
                device_id=(my,), device_id_type=_DevId.MESH,
            ).wait_recv()

        for dest in others:
            push(CT, ct_ref.at[myrows], ct_ref.at[myrows], dest)
        for dest in others:
            dcols = pl.ds(dest * GW, GW)
            push(KU, kusnd_ref.at[:, dcols], kup_ref.at[myrows], dest)
            push(VU, vusnd_ref.at[:, dcols], vup_ref.at[myrows], dest)

        def stream_chunk(w_hbm, off, slot):
            cp = pltpu.make_async_copy(
                w_hbm.at[:, pl.ds(off, CW)],
                wstage_ref.at[slot],
                dma_sems.at[slot],
            )
            cp.start()
            return cp

        cps = [stream_chunk(wq_ref, my * GW, 0),
               stream_chunk(wq_ref, my * GW + CW, 1)]
        for ci in range(2):
            cps[ci].wait()
            wb = wstage_ref[ci].astype(f)
            q_ref[:, ci * CW:(ci + 1) * CW] = (
                _dot(xv, wb, ((1,), (0,))) * SCALE).astype(f)
        myrcols = pl.ds(my * GR, GR)
        qr_ref[...] = (_dot(xv, wqr_ref[:, myrcols].astype(f), ((1,), (0,)))
                       * SCALE).astype(f)
        kr_ref[...] = _dot(xv, wkr_ref[...].astype(f), ((1,), (0,))).astype(f)

        cps = [stream_chunk(wo_ref, ci * CW, ci) for ci in range(NS)]

        for o in others:
            drain(CT, ct_ref, o)
        for o in others:
            drain(KU, kup_ref, o)
        kg_ref[...] = _dot(ct_ref[...], kup_ref[...], ((0,), (0,))).astype(f)
        for o in others:
            drain(VU, vup_ref, o)
        vg_ref[...] = _dot(ct_ref[...], vup_ref[...], ((0,), (0,))).astype(f)

        for lh in range(HL):
            cols = slice(lh * DH, (lh + 1) * DH)
            rcols = slice(lh * DR, (lh + 1) * DR)
            for b in range(B):
                rows = slice(b * S, (b + 1) * S)
                s = _dot(q_ref[rows, cols], kg_ref[rows, cols], ((1,), (1,)))
                s = s + _dot(qr_ref[rows, rcols], kr_ref[rows, :],
                             ((1,), (1,)))
                e = jnp.exp(s.astype(f))
                recip = 1.0 / jnp.sum(e, axis=-1, keepdims=True,
                                      dtype=jnp.float32)
                en = (e * recip).astype(f)
                otb = _dot(vg_ref[rows, cols], en, ((0,), (1,)))
                ot_ref[my, lh * DH:(lh + 1) * DH, rows] = otb.astype(f)
            src = ot_ref.at[my, pl.ds(lh * DH, DH), :]
            for dest in others:
                r = pltpu.make_async_remote_copy(
                    src_ref=src, dst_ref=src,
                    send_sem=osend_sems.at[dest, lh],
                    recv_sem=orecv_sems.at[my, lh],
                    device_id=(dest,), device_id_type=_DevId.MESH,
                )
                r.start()
                sends.append(r)

        for ci in range(NC):
            slot = ci % NS
            cps[slot].wait()
            wob_ref[:, ci * CW:(ci + 1) * CW] = wstage_ref[slot].astype(f)
            if ci + NS < NC:
                cps[slot] = stream_chunk(wo_ref, (ci + NS) * CW, slot)

        for b in range(B):
            bcols = pl.ds(b * S, S)
            out_ref[b, :, :] = _dot(
                ot_ref[my, :, bcols], wob_ref[pl.ds(my * GW, GW), :],
                ((0,), (0,)))
        for o in others:
            for lh in range(HL):
                src = ot_ref.at[o, pl.ds(lh * DH, DH), :]
                pltpu.make_async_remote_copy(
                    src_ref=src, dst_ref=src,
                    send_sem=osend_sems.at[o, lh],
                    recv_sem=orecv_sems.at[o, lh],
                    device_id=(my,), device_id_type=_DevId.MESH,
                ).wait_recv()
            for b in range(B):
                bcols = pl.ds(b * S, S)
                out_ref[b, :, :] += _dot(
                    ot_ref[o, :, bcols], wob_ref[pl.ds(o * GW, GW), :],
                    ((0,), (0,)))

        for r in sends:
            r.wait_send()

    vmem = pl.BlockSpec(memory_space=pltpu.VMEM)
    anym = pl.BlockSpec(memory_space=pl.ANY)
    return pl.pallas_call(
        body,
        out_shape=jax.ShapeDtypeStruct((B, S, D), jnp.float32),
        in_specs=[vmem, vmem, vmem, vmem, anym, vmem, vmem, anym],
        out_specs=vmem,
        scratch_shapes=[
            pltpu.VMEM((DC, BS), f),
            pltpu.VMEM((DC, GW), f),
            pltpu.VMEM((DC, GW), f),
            pltpu.VMEM((DCL, D), f),
            pltpu.VMEM((DCL, D), f),
            pltpu.VMEM((BS, GW), f),
            pltpu.VMEM((BS, GR), f),
            pltpu.VMEM((BS, DR), f),
            pltpu.VMEM((BS, GW), f),
            pltpu.VMEM((BS, GW), f),
            pltpu.VMEM((N_DEV, GW, BS), f),
            pltpu.VMEM((D, D), f),
            pltpu.VMEM((NS, D, CW), jnp.float32),
            pltpu.SemaphoreType.DMA((3, N_DEV)),
            pltpu.SemaphoreType.DMA((3, N_DEV)),
            pltpu.SemaphoreType.DMA((N_DEV, HL)),
            pltpu.SemaphoreType.DMA((N_DEV, HL)),
            pltpu.SemaphoreType.DMA((NS,)),
        ],
        compiler_params=_CompilerParams(
            collective_id=0, vmem_limit_bytes=64 * 1024 * 1024),
    )(x.reshape(BS, D), Wdkv, Wuk, Wuv, Wq, Wqr, Wkr, Wo)
